# baseline (speedup 1.0000x reference)
import functools

import jax
import jax.numpy as jnp
from jax import lax
from jax.experimental import pallas as pl
from jax.experimental.pallas import tpu as pltpu
from jax.experimental.pallas import tpu_sc as plsc

_NC = 2
_NS = 16
_NW = _NC * _NS

_ROWS_PER_STREAM = 128
_STREAMS_PER_SUPER = 8


def _sc_gather(table_flat, idx):
    n_rows, d = table_flat.shape
    nw, groups, rps = idx.shape
    assert nw == _NW and rps == _ROWS_PER_STREAM
    assert groups % _STREAMS_PER_SUPER == 0
    supers = groups // _STREAMS_PER_SUPER
    rows_per_super = _STREAMS_PER_SUPER * rps
    ipw = groups * rps
    total = nw * ipw

    mesh = plsc.VectorSubcoreMesh(
        core_axis_name="c", subcore_axis_name="s",
        num_cores=_NC, num_subcores=_NS)

    @functools.partial(
        pl.kernel,
        mesh=mesh,
        compiler_params=pltpu.CompilerParams(use_tc_tiling_on_sc=False),
        out_type=jax.ShapeDtypeStruct((total, d), jnp.float32),
        scratch_types=[
            pltpu.VMEM((groups, rps), jnp.int32),
            pltpu.VMEM((rows_per_super, d), jnp.float32),
            pltpu.SemaphoreType.DMA,
        ],
    )
    def gather_kernel(tbl_hbm, idx_hbm, out_hbm, idx_v, rows_v, sem):
        wid = lax.axis_index("s") * _NC + lax.axis_index("c")
        base = wid * ipw
        pltpu.sync_copy(idx_hbm.at[wid], idx_v)

        @pl.loop(0, supers)
        def _super(sp):
            cps = []
            for j in range(_STREAMS_PER_SUPER):
                g = sp * _STREAMS_PER_SUPER + j
                cps.append(pltpu.async_copy(
                    tbl_hbm.at[idx_v.at[g]],
                    rows_v.at[pl.ds(j * rps, rps)],
                    sem))
            for cp in cps:
                cp.wait()
            pltpu.sync_copy(
                rows_v, out_hbm.at[pl.ds(base + sp * rows_per_super, rows_per_super)])

    return gather_kernel(table_flat, idx)


def _mlp_body(xi_ref, xe_ref, w1a_ref, w1b_ref, w2_ref, w3_ref,
              b1_ref, b2_ref, b3_ref, o_ref):
    f_cat = xe_ref.shape[0]
    x1 = jnp.dot(xi_ref[...], w1a_ref[...], preferred_element_type=jnp.float32)
    for f in range(f_cat):
        x1 = x1 + jnp.dot(xe_ref[f], w1b_ref[f],
                          preferred_element_type=jnp.float32)
    h1 = jnp.maximum(x1 + b1_ref[...], 0.0)
    h2 = jnp.maximum(
        jnp.dot(h1, w2_ref[...], preferred_element_type=jnp.float32) + b2_ref[...], 0.0)
    o = jnp.dot(h2, w3_ref[...], preferred_element_type=jnp.float32) + b3_ref[...]
    o_ref[...] = jax.nn.sigmoid(o)


def _mlp(xi, xe3, w1a, w1b, w2, w3, b1, b2, b3, tile_b=1024):
    b, f_cont = xi.shape
    f_cat, _, d = xe3.shape
    grid = (b // tile_b,)
    full = lambda shape: pl.BlockSpec(shape, lambda i: tuple(0 for _ in shape))
    return pl.pallas_call(
        _mlp_body,
        grid=grid,
        in_specs=[
            pl.BlockSpec((tile_b, f_cont), lambda i: (i, 0)),
            pl.BlockSpec((f_cat, tile_b, d), lambda i: (0, i, 0)),
            full(w1a.shape),
            full(w1b.shape),
            full(w2.shape),
            full(w3.shape),
            full(b1.shape),
            full(b2.shape),
            full(b3.shape),
        ],
        out_specs=pl.BlockSpec((tile_b, 1), lambda i: (i, 0)),
        out_shape=jax.ShapeDtypeStruct((b, 1), jnp.float32),
    )(xi, xe3, w1a, w1b, w2, w3, b1, b2, b3)


def kernel(xi, xv, tables, W1, b1, W2, b2, W3, b3):
    b, f_cat = xv.shape
    f, v, d = tables.shape
    f_cont = xi.shape[1]
    idx = xv.astype(jnp.int32).T + (jnp.arange(f, dtype=jnp.int32) * v)[:, None]
    ipw = (b * f_cat) // _NW
    idx = idx.reshape(_NW, ipw // _ROWS_PER_STREAM, _ROWS_PER_STREAM)
    xe = _sc_gather(tables.reshape(f * v, d), idx)
    xe3 = xe.reshape(f_cat, b, d)
    return _mlp(
        xi, xe3,
        W1[:f_cont], W1[f_cont:].reshape(f_cat, d, -1), W2, W3,
        b1.reshape(1, -1), b2.reshape(1, -1), b3.reshape(1, 1))


# speedup vs baseline: 1.0726x; 1.0726x over previous
import functools

import jax
import jax.numpy as jnp
from jax import lax
from jax.experimental import pallas as pl
from jax.experimental.pallas import tpu as pltpu
from jax.experimental.pallas import tpu_sc as plsc

_NC = 2
_NS = 16
_NW = _NC * _NS

_ROWS_PER_STREAM = 128
_STREAMS_PER_SUPER = 8


def _sc_gather(table_flat, idx):
    n_rows, d = table_flat.shape
    nw, groups, rps = idx.shape
    assert nw == _NW and rps == _ROWS_PER_STREAM
    assert groups % _STREAMS_PER_SUPER == 0
    supers = groups // _STREAMS_PER_SUPER
    rows_per_super = _STREAMS_PER_SUPER * rps
    ipw = groups * rps
    total = nw * ipw

    mesh = plsc.VectorSubcoreMesh(
        core_axis_name="c", subcore_axis_name="s",
        num_cores=_NC, num_subcores=_NS)

    @functools.partial(
        pl.kernel,
        mesh=mesh,
        compiler_params=pltpu.CompilerParams(use_tc_tiling_on_sc=False),
        out_type=jax.ShapeDtypeStruct((total, d), jnp.float32),
        scratch_types=[
            pltpu.VMEM((groups, rps), jnp.int32),
            pltpu.VMEM((rows_per_super, d), jnp.float32),
            pltpu.SemaphoreType.DMA,
        ],
    )
    def gather_kernel(tbl_hbm, idx_hbm, out_hbm, idx_v, rows_v, sem):
        wid = lax.axis_index("s") * _NC + lax.axis_index("c")
        base = wid * ipw
        pltpu.sync_copy(idx_hbm.at[wid], idx_v)

        @pl.loop(0, supers)
        def _super(sp):
            cps = []
            for j in range(_STREAMS_PER_SUPER):
                g = sp * _STREAMS_PER_SUPER + j
                cps.append(pltpu.async_copy(
                    tbl_hbm.at[idx_v.at[g]],
                    rows_v.at[pl.ds(j * rps, rps)],
                    sem))
            for cp in cps:
                cp.wait()
            pltpu.sync_copy(
                rows_v, out_hbm.at[pl.ds(base + sp * rows_per_super, rows_per_super)])

    return gather_kernel(table_flat, idx)


_VC = 2048
_QR = 512


def _transpose_body(tt_ref, out_ref):
    x = tt_ref[0]
    ys = [jnp.swapaxes(x[:, a * _QR:(a + 1) * _QR], 0, 1) for a in range(4)]
    out_ref[0] = jnp.concatenate(ys, axis=1)


def _tc_transpose(tt):
    f, d, v = tt.shape
    chunks = (v + _VC - 1) // _VC
    return pl.pallas_call(
        _transpose_body,
        grid=(f, chunks),
        in_specs=[pl.BlockSpec((1, d, _VC), lambda i, j: (i, 0, j))],
        out_specs=pl.BlockSpec((1, _QR, 128), lambda i, j: (i, j, 0)),
        out_shape=jax.ShapeDtypeStruct((f, chunks * _QR, 128), jnp.float32),
    )(tt)


def _mlp_body(xi_ref, xe_ref, w1a_ref, w1b_ref, w2_ref, w3_ref,
              b1_ref, b2_ref, b3_ref, o_ref):
    f_cat = xe_ref.shape[0]
    x1 = jnp.dot(xi_ref[...], w1a_ref[...], preferred_element_type=jnp.float32)
    for f in range(f_cat):
        x1 = x1 + jnp.dot(xe_ref[f], w1b_ref[f],
                          preferred_element_type=jnp.float32)
    h1 = jnp.maximum(x1 + b1_ref[...], 0.0)
    h2 = jnp.maximum(
        jnp.dot(h1, w2_ref[...], preferred_element_type=jnp.float32) + b2_ref[...], 0.0)
    o = jnp.dot(h2, w3_ref[...], preferred_element_type=jnp.float32) + b3_ref[...]
    o_ref[...] = jax.nn.sigmoid(o)


def _mlp(xi, xe3, w1a, w1b, w2, w3, b1, b2, b3, tile_b=1024):
    b, f_cont = xi.shape
    f_cat, _, d = xe3.shape
    grid = (b // tile_b,)
    full = lambda shape: pl.BlockSpec(shape, lambda i: tuple(0 for _ in shape))
    return pl.pallas_call(
        _mlp_body,
        grid=grid,
        in_specs=[
            pl.BlockSpec((tile_b, f_cont), lambda i: (i, 0)),
            pl.BlockSpec((f_cat, tile_b, d), lambda i: (0, i, 0)),
            full(w1a.shape),
            full(w1b.shape),
            full(w2.shape),
            full(w3.shape),
            full(b1.shape),
            full(b2.shape),
            full(b3.shape),
        ],
        out_specs=pl.BlockSpec((tile_b, 1), lambda i: (i, 0)),
        out_shape=jax.ShapeDtypeStruct((b, 1), jnp.float32),
    )(xi, xe3, w1a, w1b, w2, w3, b1, b2, b3)


def kernel(xi, xv, tables, W1, b1, W2, b2, W3, b3):
    b, f_cat = xv.shape
    f, v, d = tables.shape
    f_cont = xi.shape[1]
    tbl_packed = _tc_transpose(jnp.swapaxes(tables, 1, 2))
    rows_pf = tbl_packed.shape[1]
    xvt = xv.astype(jnp.int32).T
    fid = jnp.arange(f, dtype=jnp.int32)[:, None]
    idx = (fid * rows_pf + (xvt >> 11) * _QR + (xvt & (_QR - 1))) * 4 \
        + ((xvt >> 9) & 3)
    ipw = (b * f_cat) // _NW
    idx = idx.reshape(_NW, ipw // _ROWS_PER_STREAM, _ROWS_PER_STREAM)
    xe = _sc_gather(tbl_packed.reshape(f * rows_pf * 4, d), idx)
    xe3 = xe.reshape(f_cat, b, d)
    return _mlp(
        xi, xe3,
        W1[:f_cont], W1[f_cont:].reshape(f_cat, d, -1), W2, W3,
        b1.reshape(1, -1), b2.reshape(1, -1), b3.reshape(1, 1))


# speedup vs baseline: 1.2980x; 1.2101x over previous
import functools

import jax
import jax.numpy as jnp
from jax import lax
from jax.experimental import pallas as pl
from jax.experimental.pallas import tpu as pltpu
from jax.experimental.pallas import tpu_sc as plsc

_NC = 2
_NS = 16
_NW = _NC * _NS

_ROWS_PER_STREAM = 128
_STREAMS_PER_SUPER = 8


def _sc_gather(table_flat, idx):
    n_rows, d = table_flat.shape
    nw, groups, rps = idx.shape
    assert nw == _NW and rps == _ROWS_PER_STREAM
    assert groups % _STREAMS_PER_SUPER == 0
    supers = groups // _STREAMS_PER_SUPER
    rows_per_super = _STREAMS_PER_SUPER * rps
    ipw = groups * rps
    total = nw * ipw

    mesh = plsc.VectorSubcoreMesh(
        core_axis_name="c", subcore_axis_name="s",
        num_cores=_NC, num_subcores=_NS)

    @functools.partial(
        pl.kernel,
        mesh=mesh,
        compiler_params=pltpu.CompilerParams(use_tc_tiling_on_sc=False),
        out_type=jax.ShapeDtypeStruct((total, d), jnp.float32),
        scratch_types=[
            pltpu.VMEM((groups, rps), jnp.int32),
            pltpu.VMEM((rows_per_super, d), jnp.float32),
            pltpu.SemaphoreType.DMA,
        ],
    )
    def gather_kernel(tbl_hbm, idx_hbm, out_hbm, idx_v, rows_v, sem):
        wid = lax.axis_index("s") * _NC + lax.axis_index("c")
        base = wid * ipw
        pltpu.sync_copy(idx_hbm.at[wid], idx_v)

        @pl.loop(0, supers)
        def _super(sp):
            cps = []
            for j in range(_STREAMS_PER_SUPER):
                g = sp * _STREAMS_PER_SUPER + j
                cps.append(pltpu.async_copy(
                    tbl_hbm.at[idx_v.at[g]],
                    rows_v.at[pl.ds(j * rps, rps)],
                    sem))
            for cp in cps:
                cp.wait()
            pltpu.sync_copy(
                rows_v, out_hbm.at[pl.ds(base + sp * rows_per_super, rows_per_super)])

    return gather_kernel(table_flat, idx)


_VC = 2048
_QR = 512


def _transpose_body(tt_ref, out_ref):
    x = tt_ref[0]
    big = jnp.concatenate(
        [x[:, a * _QR:(a + 1) * _QR] for a in range(4)], axis=0)
    out_ref[0] = jnp.swapaxes(big, 0, 1)


def _tc_transpose(tt):
    f, d, v = tt.shape
    chunks = (v + _VC - 1) // _VC
    return pl.pallas_call(
        _transpose_body,
        grid=(f, chunks),
        in_specs=[pl.BlockSpec((1, d, _VC), lambda i, j: (i, 0, j))],
        out_specs=pl.BlockSpec((1, _QR, 128), lambda i, j: (i, j, 0)),
        out_shape=jax.ShapeDtypeStruct((f, chunks * _QR, 128), jnp.float32),
    )(tt)


def _mlp_body(xi_ref, xe_ref, w1a_ref, w1b_ref, w2_ref, w3_ref,
              b1_ref, b2_ref, b3_ref, o_ref):
    f_cat = xe_ref.shape[0]
    x1 = jnp.dot(xi_ref[...], w1a_ref[...], preferred_element_type=jnp.float32)
    for f in range(f_cat):
        x1 = x1 + jnp.dot(xe_ref[f], w1b_ref[f],
                          preferred_element_type=jnp.float32)
    h1 = jnp.maximum(x1 + b1_ref[...], 0.0)
    h2 = jnp.maximum(
        jnp.dot(h1, w2_ref[...], preferred_element_type=jnp.float32) + b2_ref[...], 0.0)
    o = jnp.dot(h2, w3_ref[...], preferred_element_type=jnp.float32) + b3_ref[...]
    o_ref[...] = jax.nn.sigmoid(o)


def _mlp(xi, xe3, w1a, w1b, w2, w3, b1, b2, b3, tile_b=1024):
    b, f_cont = xi.shape
    f_cat, _, d = xe3.shape
    grid = (b // tile_b,)
    full = lambda shape: pl.BlockSpec(shape, lambda i: tuple(0 for _ in shape))
    return pl.pallas_call(
        _mlp_body,
        grid=grid,
        in_specs=[
            pl.BlockSpec((tile_b, f_cont), lambda i: (i, 0)),
            pl.BlockSpec((f_cat, tile_b, d), lambda i: (0, i, 0)),
            full(w1a.shape),
            full(w1b.shape),
            full(w2.shape),
            full(w3.shape),
            full(b1.shape),
            full(b2.shape),
            full(b3.shape),
        ],
        out_specs=pl.BlockSpec((tile_b, 1), lambda i: (i, 0)),
        out_shape=jax.ShapeDtypeStruct((b, 1), jnp.float32),
    )(xi, xe3, w1a, w1b, w2, w3, b1, b2, b3)


def kernel(xi, xv, tables, W1, b1, W2, b2, W3, b3):
    b, f_cat = xv.shape
    f, v, d = tables.shape
    f_cont = xi.shape[1]
    tbl_packed = _tc_transpose(jnp.swapaxes(tables, 1, 2))
    rows_pf = tbl_packed.shape[1]
    xvt = xv.astype(jnp.int32).T
    fid = jnp.arange(f, dtype=jnp.int32)[:, None]
    idx = (fid * rows_pf + (xvt >> 11) * _QR + (xvt & (_QR - 1))) * 4 \
        + ((xvt >> 9) & 3)
    ipw = (b * f_cat) // _NW
    idx = idx.reshape(_NW, ipw // _ROWS_PER_STREAM, _ROWS_PER_STREAM)
    xe = _sc_gather(tbl_packed.reshape(f * rows_pf * 4, d), idx)
    xe3 = xe.reshape(f_cat, b, d)
    return _mlp(
        xi, xe3,
        W1[:f_cont], W1[f_cont:].reshape(f_cat, d, -1), W2, W3,
        b1.reshape(1, -1), b2.reshape(1, -1), b3.reshape(1, 1))


# speedup vs baseline: 2.1913x; 1.6882x over previous
import functools

import jax
import jax.numpy as jnp
from jax import lax
from jax.experimental import pallas as pl
from jax.experimental.pallas import tpu as pltpu
from jax.experimental.pallas import tpu_sc as plsc

_NC = 2
_NS = 16
_NW = _NC * _NS

_ROWS_PER_STREAM = 128
_STREAMS_PER_SUPER = 8


def _sc_gather(table_flat, idx):
    n_rows, d = table_flat.shape
    nw, groups, rps = idx.shape
    assert nw == _NW and rps == _ROWS_PER_STREAM
    assert groups % _STREAMS_PER_SUPER == 0
    supers = groups // _STREAMS_PER_SUPER
    rows_per_super = _STREAMS_PER_SUPER * rps
    ipw = groups * rps
    total = nw * ipw

    mesh = plsc.VectorSubcoreMesh(
        core_axis_name="c", subcore_axis_name="s",
        num_cores=_NC, num_subcores=_NS)

    @functools.partial(
        pl.kernel,
        mesh=mesh,
        compiler_params=pltpu.CompilerParams(use_tc_tiling_on_sc=False),
        out_type=jax.ShapeDtypeStruct((total, d), jnp.float32),
        scratch_types=[
            pltpu.VMEM((groups, rps), jnp.int32),
            pltpu.VMEM((rows_per_super, d), jnp.float32),
            pltpu.SemaphoreType.DMA,
        ],
    )
    def gather_kernel(tbl_hbm, idx_hbm, out_hbm, idx_v, rows_v, sem):
        wid = lax.axis_index("s") * _NC + lax.axis_index("c")
        base = wid * ipw
        pltpu.sync_copy(idx_hbm.at[wid], idx_v)

        @pl.loop(0, supers)
        def _super(sp):
            cps = []
            for j in range(_STREAMS_PER_SUPER):
                g = sp * _STREAMS_PER_SUPER + j
                cps.append(pltpu.async_copy(
                    tbl_hbm.at[idx_v.at[g]],
                    rows_v.at[pl.ds(j * rps, rps)],
                    sem))
            for cp in cps:
                cp.wait()
            pltpu.sync_copy(
                rows_v, out_hbm.at[pl.ds(base + sp * rows_per_super, rows_per_super)])

    return gather_kernel(table_flat, idx)


_VC = 2048
_QR = 512


_CB = 4


def _transpose_body(tt_ref, out_ref):
    x = tt_ref[0]
    for c in range(_CB):
        big = jnp.concatenate(
            [x[:, c * _VC + a * _QR: c * _VC + (a + 1) * _QR]
             for a in range(4)], axis=0)
        out_ref[0, c * _QR:(c + 1) * _QR, :] = jnp.swapaxes(big, 0, 1)


def _tc_transpose(tt):
    f, d, v = tt.shape
    steps = (v + _CB * _VC - 1) // (_CB * _VC)
    chunks = steps * _CB
    return pl.pallas_call(
        _transpose_body,
        grid=(f, steps),
        in_specs=[pl.BlockSpec((1, d, _CB * _VC), lambda i, j: (i, 0, j))],
        out_specs=pl.BlockSpec((1, _CB * _QR, 128), lambda i, j: (i, j, 0)),
        out_shape=jax.ShapeDtypeStruct((f, chunks * _QR, 128), jnp.float32),
    )(tt)


def _mlp_body(xi_ref, xe_ref, w1a_ref, w1b_ref, w2_ref, w3_ref,
              b1_ref, b2_ref, b3_ref, o_ref):
    f_cat = xe_ref.shape[0]
    x1 = jnp.dot(xi_ref[...], w1a_ref[...], preferred_element_type=jnp.float32)
    for f in range(f_cat):
        x1 = x1 + jnp.dot(xe_ref[f], w1b_ref[f],
                          preferred_element_type=jnp.float32)
    h1 = jnp.maximum(x1 + b1_ref[...], 0.0)
    h2 = jnp.maximum(
        jnp.dot(h1, w2_ref[...], preferred_element_type=jnp.float32) + b2_ref[...], 0.0)
    o = jnp.dot(h2, w3_ref[...], preferred_element_type=jnp.float32) + b3_ref[...]
    o_ref[...] = jax.nn.sigmoid(o)


def _mlp(xi, xe3, w1a, w1b, w2, w3, b1, b2, b3, tile_b=1024):
    b, f_cont = xi.shape
    f_cat, _, d = xe3.shape
    grid = (b // tile_b,)
    full = lambda shape: pl.BlockSpec(shape, lambda i: tuple(0 for _ in shape))
    return pl.pallas_call(
        _mlp_body,
        grid=grid,
        in_specs=[
            pl.BlockSpec((tile_b, f_cont), lambda i: (i, 0)),
            pl.BlockSpec((f_cat, tile_b, d), lambda i: (0, i, 0)),
            full(w1a.shape),
            full(w1b.shape),
            full(w2.shape),
            full(w3.shape),
            full(b1.shape),
            full(b2.shape),
            full(b3.shape),
        ],
        out_specs=pl.BlockSpec((tile_b, 1), lambda i: (i, 0)),
        out_shape=jax.ShapeDtypeStruct((b, 1), jnp.float32),
    )(xi, xe3, w1a, w1b, w2, w3, b1, b2, b3)


def kernel(xi, xv, tables, W1, b1, W2, b2, W3, b3):
    b, f_cat = xv.shape
    f, v, d = tables.shape
    f_cont = xi.shape[1]
    tbl_packed = _tc_transpose(jnp.swapaxes(tables, 1, 2))
    rows_pf = tbl_packed.shape[1]
    xvt = xv.astype(jnp.int32).T
    fid = jnp.arange(f, dtype=jnp.int32)[:, None]
    idx = (fid * rows_pf + (xvt >> 11) * _QR + (xvt & (_QR - 1))) * 4 \
        + ((xvt >> 9) & 3)
    ipw = (b * f_cat) // _NW
    idx = idx.reshape(_NW, ipw // _ROWS_PER_STREAM, _ROWS_PER_STREAM)
    xe = _sc_gather(tbl_packed.reshape(f * rows_pf * 4, d), idx)
    xe3 = xe.reshape(f_cat, b, d)
    return _mlp(
        xi, xe3,
        W1[:f_cont], W1[f_cont:].reshape(f_cat, d, -1), W2, W3,
        b1.reshape(1, -1), b2.reshape(1, -1), b3.reshape(1, 1))


# speedup vs baseline: 2.9159x; 1.3306x over previous
import functools

import jax
import jax.numpy as jnp
from jax import lax
from jax.experimental import pallas as pl
from jax.experimental.pallas import tpu as pltpu
from jax.experimental.pallas import tpu_sc as plsc

_NC = 2
_NS = 16
_NW = _NC * _NS

_ROWS_PER_STREAM = 128
_STREAMS_PER_SUPER = 8


def _sc_gather(table_flat, idx):
    n_rows, d = table_flat.shape
    nw, groups, rps = idx.shape
    assert nw == _NW and rps == _ROWS_PER_STREAM
    assert groups % _STREAMS_PER_SUPER == 0
    supers = groups // _STREAMS_PER_SUPER
    rows_per_super = _STREAMS_PER_SUPER * rps
    ipw = groups * rps
    total = nw * ipw

    mesh = plsc.VectorSubcoreMesh(
        core_axis_name="c", subcore_axis_name="s",
        num_cores=_NC, num_subcores=_NS)

    @functools.partial(
        pl.kernel,
        mesh=mesh,
        compiler_params=pltpu.CompilerParams(use_tc_tiling_on_sc=False),
        out_type=jax.ShapeDtypeStruct((total, d), jnp.float32),
        scratch_types=[
            pltpu.VMEM((groups, rps), jnp.int32),
            pltpu.VMEM((rows_per_super, d), jnp.float32),
            pltpu.SemaphoreType.DMA,
        ],
    )
    def gather_kernel(tbl_hbm, idx_hbm, out_hbm, idx_v, rows_v, sem):
        wid = lax.axis_index("s") * _NC + lax.axis_index("c")
        base = wid * ipw
        pltpu.sync_copy(idx_hbm.at[wid], idx_v)

        @pl.loop(0, supers)
        def _super(sp):
            cps = []
            for j in range(_STREAMS_PER_SUPER):
                g = sp * _STREAMS_PER_SUPER + j
                cps.append(pltpu.async_copy(
                    tbl_hbm.at[idx_v.at[g]],
                    rows_v.at[pl.ds(j * rps, rps)],
                    sem))
            for cp in cps:
                cp.wait()
            pltpu.sync_copy(
                rows_v, out_hbm.at[pl.ds(base + sp * rows_per_super, rows_per_super)])

    return gather_kernel(table_flat, idx)


_VC = 2048
_QR = 512


_CB = 4


def _transpose_body(tt_ref, out_ref):
    x = tt_ref[0]
    for c in range(_CB):
        big = jnp.concatenate(
            [x[:, c * _VC + a * _QR: c * _VC + (a + 1) * _QR]
             for a in range(4)], axis=0)
        out_ref[0, c * _QR:(c + 1) * _QR, :] = jnp.swapaxes(big, 0, 1)


def _tc_transpose(tt):
    f, d, v = tt.shape
    steps = (v + _CB * _VC - 1) // (_CB * _VC)
    chunks = steps * _CB
    return pl.pallas_call(
        _transpose_body,
        grid=(f, steps),
        in_specs=[pl.BlockSpec((1, d, _CB * _VC), lambda i, j: (i, 0, j))],
        out_specs=pl.BlockSpec((1, _CB * _QR, 128), lambda i, j: (i, j, 0)),
        out_shape=jax.ShapeDtypeStruct((f, chunks * _QR, 128), jnp.float32),
    )(tt)


def _mlp_body(xi_ref, xe_ref, w1a_ref, w1b_ref, w2_ref, w3_ref,
              b1_ref, b2_ref, b3_ref, o_ref):
    f_cat = xe_ref.shape[0]
    z = jnp.dot(xi_ref[...], w1a_ref[...], preferred_element_type=jnp.float32)
    for f in range(f_cat):
        z = z + jnp.dot(xe_ref[f], w1b_ref[f],
                        preferred_element_type=jnp.float32)
    h1 = jnp.maximum(z + b1_ref[...], 0.0)
    h2 = jnp.maximum(
        jnp.dot(h1, w2_ref[...], preferred_element_type=jnp.float32) + b2_ref[...], 0.0)
    o = jnp.dot(h2, w3_ref[...], preferred_element_type=jnp.float32) + b3_ref[...]
    o_ref[...] = jax.nn.sigmoid(o)


def _mlp_packed(xi_p, xe_p, w1a_p, w1b_p, w2_p, w3_p, b1_p, b2_p, b3_p,
                tile_r=256):
    r_tot = xi_p.shape[0]
    f_cat = xe_p.shape[0]
    grid = (r_tot // tile_r,)
    full = lambda shape: pl.BlockSpec(shape, lambda i: tuple(0 for _ in shape))
    return pl.pallas_call(
        _mlp_body,
        grid=grid,
        in_specs=[
            pl.BlockSpec((tile_r, xi_p.shape[1]), lambda i: (i, 0)),
            pl.BlockSpec((f_cat, tile_r, 128), lambda i: (0, i, 0)),
            full(w1a_p.shape),
            full(w1b_p.shape),
            full(w2_p.shape),
            full(w3_p.shape),
            full(b1_p.shape),
            full(b2_p.shape),
            full(b3_p.shape),
        ],
        out_specs=pl.BlockSpec((tile_r, 4), lambda i: (i, 0)),
        out_shape=jax.ShapeDtypeStruct((r_tot, 4), jnp.float32),
    )(xi_p, xe_p, w1a_p, w1b_p, w2_p, w3_p, b1_p, b2_p, b3_p)


def kernel(xi, xv, tables, W1, b1, W2, b2, W3, b3):
    b, f_cat = xv.shape
    f, v, d = tables.shape
    f_cont = xi.shape[1]
    tbl_packed = _tc_transpose(jnp.swapaxes(tables, 1, 2))
    rows_pf = tbl_packed.shape[1]
    xvt = xv.astype(jnp.int32).T
    fid = jnp.arange(f, dtype=jnp.int32)[:, None]
    idx = (fid * rows_pf + (xvt >> 11) * _QR + (xvt & (_QR - 1))) * 4 \
        + ((xvt >> 9) & 3)
    ipw = (b * f_cat) // _NW
    idx = idx.reshape(_NW, ipw // _ROWS_PER_STREAM, _ROWS_PER_STREAM)
    xe = _sc_gather(tbl_packed.reshape(f * rows_pf * 4, d), idx)
    xe_p = xe.reshape(f_cat, b // 4, 4 * d)
    xi_p = xi.reshape(b // 4, 4 * f_cont)
    eye4 = jnp.eye(4, dtype=jnp.float32)
    w1a = W1[:f_cont]
    w1b3 = W1[f_cont:].reshape(f_cat, d, -1)
    h1 = W1.shape[1]
    w1a_p = jnp.einsum("xy,cj->xcyj", eye4, w1a).reshape(4 * f_cont, 4 * h1)
    w1b_p = jnp.einsum("xy,fdj->fxdyj", eye4, w1b3).reshape(f_cat, 4 * d, 4 * h1)
    w2_p = jnp.einsum("xy,ij->xiyj", eye4, W2).reshape(4 * h1, 4 * W2.shape[1])
    w3_p = jnp.einsum("xy,j->xjy", eye4, W3[:, 0]).reshape(4 * W3.shape[0], 4)
    b1_p = jnp.tile(b1, 4).reshape(1, -1)
    b2_p = jnp.tile(b2, 4).reshape(1, -1)
    b3_p = b3.reshape(1, 1)
    o_p = _mlp_packed(xi_p, xe_p, w1a_p, w1b_p, w2_p, w3_p, b1_p, b2_p, b3_p)
    return o_p.reshape(b, 1)


# speedup vs baseline: 3.6648x; 1.2568x over previous
import functools

import jax
import jax.numpy as jnp
from jax import lax
from jax.experimental import pallas as pl
from jax.experimental.pallas import tpu as pltpu
from jax.experimental.pallas import tpu_sc as plsc

_NC = 2
_NS = 16
_NW = _NC * _NS

_ROWS_PER_STREAM = 128
_STREAMS_PER_SUPER = 8


def _sc_gather(table_flat, idx):
    n_rows, d = table_flat.shape
    nw, groups, rps = idx.shape
    assert nw == _NW and rps == _ROWS_PER_STREAM
    assert groups % _STREAMS_PER_SUPER == 0
    supers = groups // _STREAMS_PER_SUPER
    rows_per_super = _STREAMS_PER_SUPER * rps
    ipw = groups * rps
    total = nw * ipw

    mesh = plsc.VectorSubcoreMesh(
        core_axis_name="c", subcore_axis_name="s",
        num_cores=_NC, num_subcores=_NS)

    @functools.partial(
        pl.kernel,
        mesh=mesh,
        compiler_params=pltpu.CompilerParams(use_tc_tiling_on_sc=False),
        out_type=jax.ShapeDtypeStruct((total, d), table_flat.dtype),
        scratch_types=[
            pltpu.VMEM((groups, rps), jnp.int32),
            pltpu.VMEM((rows_per_super, d), table_flat.dtype),
            pltpu.SemaphoreType.DMA,
        ],
    )
    def gather_kernel(tbl_hbm, idx_hbm, out_hbm, idx_v, rows_v, sem):
        wid = lax.axis_index("s") * _NC + lax.axis_index("c")
        base = wid * ipw
        pltpu.sync_copy(idx_hbm.at[wid], idx_v)

        @pl.loop(0, supers)
        def _super(sp):
            cps = []
            for j in range(_STREAMS_PER_SUPER):
                g = sp * _STREAMS_PER_SUPER + j
                cps.append(pltpu.async_copy(
                    tbl_hbm.at[idx_v.at[g]],
                    rows_v.at[pl.ds(j * rps, rps)],
                    sem))
            for cp in cps:
                cp.wait()
            pltpu.sync_copy(
                rows_v, out_hbm.at[pl.ds(base + sp * rows_per_super, rows_per_super)])

    return gather_kernel(table_flat, idx)


_VC = 2048
_QR = 512


_CB = 4


_FB = 2


def _transpose_body(tt_ref, out_ref):
    for g in range(tt_ref.shape[0]):
        x = tt_ref[g]
        for c in range(_CB):
            big = jnp.concatenate(
                [x[:, c * _VC + a * _QR: c * _VC + (a + 1) * _QR]
                 for a in range(4)], axis=0)
            out_ref[g, c * _QR:(c + 1) * _QR, :] = jnp.swapaxes(big, 0, 1)


def _tc_transpose(tt):
    f, d, v = tt.shape
    fb = _FB if f % _FB == 0 else 1
    steps = (v + _CB * _VC - 1) // (_CB * _VC)
    chunks = steps * _CB
    return pl.pallas_call(
        _transpose_body,
        grid=(f // fb, steps),
        in_specs=[pl.BlockSpec((fb, d, _CB * _VC), lambda i, j: (i, 0, j))],
        out_specs=pl.BlockSpec((fb, _CB * _QR, 128), lambda i, j: (i, j, 0)),
        out_shape=jax.ShapeDtypeStruct((f, chunks * _QR, 128), jnp.float32),
    )(tt)


def _mlp_body(xi_ref, xe_ref, w1a_ref, w1b_ref, w2_ref, w3_ref,
              b1_ref, b2_ref, b3_ref, o_ref):
    f_cat = xe_ref.shape[0]
    z = jnp.dot(xi_ref[...], w1a_ref[...], preferred_element_type=jnp.float32)
    for f in range(f_cat):
        z = z + jnp.dot(xe_ref[f], w1b_ref[f],
                        preferred_element_type=jnp.float32)
    h1 = jnp.maximum(z + b1_ref[...], 0.0)
    h2 = jnp.maximum(
        jnp.dot(h1, w2_ref[...], preferred_element_type=jnp.float32) + b2_ref[...], 0.0)
    o = jnp.dot(h2, w3_ref[...], preferred_element_type=jnp.float32) + b3_ref[...]
    o_ref[...] = jax.nn.sigmoid(o)


def _mlp_packed(xi_p, xe_p, w1a_p, w1b_p, w2_p, w3_p, b1_p, b2_p, b3_p,
                tile_r=256):
    r_tot = xi_p.shape[0]
    f_cat = xe_p.shape[0]
    grid = (r_tot // tile_r,)
    full = lambda shape: pl.BlockSpec(shape, lambda i: tuple(0 for _ in shape))
    return pl.pallas_call(
        _mlp_body,
        grid=grid,
        in_specs=[
            pl.BlockSpec((tile_r, xi_p.shape[1]), lambda i: (i, 0)),
            pl.BlockSpec((f_cat, tile_r, 128), lambda i: (0, i, 0)),
            full(w1a_p.shape),
            full(w1b_p.shape),
            full(w2_p.shape),
            full(w3_p.shape),
            full(b1_p.shape),
            full(b2_p.shape),
            full(b3_p.shape),
        ],
        out_specs=pl.BlockSpec((tile_r, 4), lambda i: (i, 0)),
        out_shape=jax.ShapeDtypeStruct((r_tot, 4), jnp.float32),
    )(xi_p, xe_p, w1a_p, w1b_p, w2_p, w3_p, b1_p, b2_p, b3_p)


def kernel(xi, xv, tables, W1, b1, W2, b2, W3, b3):
    b, f_cat = xv.shape
    f, v, d = tables.shape
    f_cont = xi.shape[1]
    tbl_packed = _tc_transpose(jnp.swapaxes(tables, 1, 2))
    rows_pf = tbl_packed.shape[1]
    xvt = xv.astype(jnp.int32).T
    fid = jnp.arange(f, dtype=jnp.int32)[:, None]
    idx = (fid * rows_pf + (xvt >> 11) * _QR + (xvt & (_QR - 1))) * 4 \
        + ((xvt >> 9) & 3)
    ipw = (b * f_cat) // _NW
    idx = idx.reshape(_NW, ipw // _ROWS_PER_STREAM, _ROWS_PER_STREAM)
    xe = _sc_gather(tbl_packed.reshape(f * rows_pf * 4, d), idx)
    xe_p = xe.reshape(f_cat, b // 4, 4 * d)
    xi_p = xi.reshape(b // 4, 4 * f_cont)
    eye4 = jnp.eye(4, dtype=jnp.float32)
    w1a = W1[:f_cont]
    w1b3 = W1[f_cont:].reshape(f_cat, d, -1)
    h1 = W1.shape[1]
    w1a_p = jnp.einsum("xy,cj->xcyj", eye4, w1a).reshape(4 * f_cont, 4 * h1)
    w1b_p = jnp.einsum("xy,fdj->fxdyj", eye4, w1b3).reshape(f_cat, 4 * d, 4 * h1)
    w2_p = jnp.einsum("xy,ij->xiyj", eye4, W2).reshape(4 * h1, 4 * W2.shape[1])
    w3_p = jnp.einsum("xy,j->xjy", eye4, W3[:, 0]).reshape(4 * W3.shape[0], 4)
    b1_p = jnp.tile(b1, 4).reshape(1, -1)
    b2_p = jnp.tile(b2, 4).reshape(1, -1)
    b3_p = b3.reshape(1, 1)
    o_p = _mlp_packed(xi_p, xe_p, w1a_p, w1b_p, w2_p, w3_p, b1_p, b2_p, b3_p)
    return o_p.reshape(b, 1)


# speedup vs baseline: 4.2516x; 1.1601x over previous
import functools

import jax
import jax.numpy as jnp
from jax import lax
from jax.experimental import pallas as pl
from jax.experimental.pallas import tpu as pltpu
from jax.experimental.pallas import tpu_sc as plsc

_NC = 2
_NS = 16
_NW = _NC * _NS

_ROWS_PER_STREAM = 128
_STREAMS_PER_SUPER = 8


def _sc_gather(table_flat, idx):
    n_rows, d = table_flat.shape
    nw, groups, rps = idx.shape
    assert nw == _NW and rps == _ROWS_PER_STREAM
    assert groups % _STREAMS_PER_SUPER == 0
    supers = groups // _STREAMS_PER_SUPER
    rows_per_super = _STREAMS_PER_SUPER * rps
    ipw = groups * rps
    total = nw * ipw

    mesh = plsc.VectorSubcoreMesh(
        core_axis_name="c", subcore_axis_name="s",
        num_cores=_NC, num_subcores=_NS)

    @functools.partial(
        pl.kernel,
        mesh=mesh,
        compiler_params=pltpu.CompilerParams(use_tc_tiling_on_sc=False),
        out_type=jax.ShapeDtypeStruct((total, d), table_flat.dtype),
        scratch_types=[
            pltpu.VMEM((groups, rps), jnp.int32),
            pltpu.VMEM((rows_per_super, d), table_flat.dtype),
            pltpu.SemaphoreType.DMA,
        ],
    )
    def gather_kernel(tbl_hbm, idx_hbm, out_hbm, idx_v, rows_v, sem):
        wid = lax.axis_index("s") * _NC + lax.axis_index("c")
        base = wid * ipw
        pltpu.sync_copy(idx_hbm.at[wid], idx_v)

        @pl.loop(0, supers)
        def _super(sp):
            cps = []
            for j in range(_STREAMS_PER_SUPER):
                g = sp * _STREAMS_PER_SUPER + j
                cps.append(pltpu.async_copy(
                    tbl_hbm.at[idx_v.at[g]],
                    rows_v.at[pl.ds(j * rps, rps)],
                    sem))
            for cp in cps:
                cp.wait()
            pltpu.sync_copy(
                rows_v, out_hbm.at[pl.ds(base + sp * rows_per_super, rows_per_super)])

    return gather_kernel(table_flat, idx)


_VC = 2048
_QR = 512


_CB = 4


_FB = 13


def _transpose_body(tt_ref, out_ref):
    for g in range(tt_ref.shape[0]):
        x = tt_ref[g]
        for c in range(_CB):
            big = jnp.concatenate(
                [x[:, c * _VC + a * _QR: c * _VC + (a + 1) * _QR]
                 for a in range(4)], axis=0)
            out_ref[g, c * _QR:(c + 1) * _QR, :] = jnp.swapaxes(big, 0, 1)


def _tc_transpose(tt):
    f, d, v = tt.shape
    fb = _FB if f % _FB == 0 else 1
    steps = (v + _CB * _VC - 1) // (_CB * _VC)
    chunks = steps * _CB
    return pl.pallas_call(
        _transpose_body,
        grid=(f // fb, steps),
        in_specs=[pl.BlockSpec((fb, d, _CB * _VC), lambda i, j: (i, 0, j))],
        out_specs=pl.BlockSpec((fb, _CB * _QR, 128), lambda i, j: (i, j, 0)),
        out_shape=jax.ShapeDtypeStruct((f, chunks * _QR, 128), jnp.float32),
    )(tt)


def _mlp_body(xi_ref, xe_ref, w1a_ref, w1b_ref, w2_ref, w3_ref,
              b1_ref, b2_ref, b3_ref, o_ref):
    f_cat = xe_ref.shape[0]
    z = jnp.dot(xi_ref[...], w1a_ref[...], preferred_element_type=jnp.float32)
    for f in range(f_cat):
        z = z + jnp.dot(xe_ref[f], w1b_ref[f],
                        preferred_element_type=jnp.float32)
    h1 = jnp.maximum(z + b1_ref[...], 0.0)
    h2 = jnp.maximum(
        jnp.dot(h1, w2_ref[...], preferred_element_type=jnp.float32) + b2_ref[...], 0.0)
    o = jnp.dot(h2, w3_ref[...], preferred_element_type=jnp.float32) + b3_ref[...]
    o_ref[...] = jax.nn.sigmoid(o)


def _mlp_packed(xi_p, xe_p, w1a_p, w1b_p, w2_p, w3_p, b1_p, b2_p, b3_p,
                tile_r=256):
    r_tot = xi_p.shape[0]
    f_cat = xe_p.shape[0]
    grid = (r_tot // tile_r,)
    full = lambda shape: pl.BlockSpec(shape, lambda i: tuple(0 for _ in shape))
    return pl.pallas_call(
        _mlp_body,
        grid=grid,
        in_specs=[
            pl.BlockSpec((tile_r, xi_p.shape[1]), lambda i: (i, 0)),
            pl.BlockSpec((f_cat, tile_r, 128), lambda i: (0, i, 0)),
            full(w1a_p.shape),
            full(w1b_p.shape),
            full(w2_p.shape),
            full(w3_p.shape),
            full(b1_p.shape),
            full(b2_p.shape),
            full(b3_p.shape),
        ],
        out_specs=pl.BlockSpec((tile_r, 4), lambda i: (i, 0)),
        out_shape=jax.ShapeDtypeStruct((r_tot, 4), jnp.float32),
    )(xi_p, xe_p, w1a_p, w1b_p, w2_p, w3_p, b1_p, b2_p, b3_p)


def kernel(xi, xv, tables, W1, b1, W2, b2, W3, b3):
    b, f_cat = xv.shape
    f, v, d = tables.shape
    f_cont = xi.shape[1]
    tbl_packed = _tc_transpose(jnp.swapaxes(tables, 1, 2))
    rows_pf = tbl_packed.shape[1]
    xvt = xv.astype(jnp.int32).T
    fid = jnp.arange(f, dtype=jnp.int32)[:, None]
    idx = (fid * rows_pf + (xvt >> 11) * _QR + (xvt & (_QR - 1))) * 4 \
        + ((xvt >> 9) & 3)
    ipw = (b * f_cat) // _NW
    idx = idx.reshape(_NW, ipw // _ROWS_PER_STREAM, _ROWS_PER_STREAM)
    xe = _sc_gather(tbl_packed.reshape(f * rows_pf * 4, d), idx)
    xe_p = xe.reshape(f_cat, b // 4, 4 * d)
    xi_p = xi.reshape(b // 4, 4 * f_cont)
    eye4 = jnp.eye(4, dtype=jnp.float32)
    w1a = W1[:f_cont]
    w1b3 = W1[f_cont:].reshape(f_cat, d, -1)
    h1 = W1.shape[1]
    w1a_p = jnp.einsum("xy,cj->xcyj", eye4, w1a).reshape(4 * f_cont, 4 * h1)
    w1b_p = jnp.einsum("xy,fdj->fxdyj", eye4, w1b3).reshape(f_cat, 4 * d, 4 * h1)
    w2_p = jnp.einsum("xy,ij->xiyj", eye4, W2).reshape(4 * h1, 4 * W2.shape[1])
    w3_p = jnp.einsum("xy,j->xjy", eye4, W3[:, 0]).reshape(4 * W3.shape[0], 4)
    b1_p = jnp.tile(b1, 4).reshape(1, -1)
    b2_p = jnp.tile(b2, 4).reshape(1, -1)
    b3_p = b3.reshape(1, 1)
    o_p = _mlp_packed(xi_p, xe_p, w1a_p, w1b_p, w2_p, w3_p, b1_p, b2_p, b3_p)
    return o_p.reshape(b, 1)


# speedup vs baseline: 4.2834x; 1.0075x over previous
import functools

import jax
import jax.numpy as jnp
from jax import lax
from jax.experimental import pallas as pl
from jax.experimental.pallas import tpu as pltpu
from jax.experimental.pallas import tpu_sc as plsc

_NC = 2
_NS = 16
_NW = _NC * _NS

_ROWS_PER_STREAM = 128
_STREAMS_PER_SUPER = 8


def _sc_gather(table_flat, idx):
    n_rows, d = table_flat.shape
    nw, groups, rps = idx.shape
    assert nw == _NW and rps == _ROWS_PER_STREAM
    assert groups % _STREAMS_PER_SUPER == 0
    supers = groups // _STREAMS_PER_SUPER
    rows_per_super = _STREAMS_PER_SUPER * rps
    ipw = groups * rps
    total = nw * ipw

    mesh = plsc.VectorSubcoreMesh(
        core_axis_name="c", subcore_axis_name="s",
        num_cores=_NC, num_subcores=_NS)

    @functools.partial(
        pl.kernel,
        mesh=mesh,
        compiler_params=pltpu.CompilerParams(use_tc_tiling_on_sc=False),
        out_type=jax.ShapeDtypeStruct((total, d), table_flat.dtype),
        scratch_types=[
            pltpu.VMEM((groups, rps), jnp.int32),
            pltpu.VMEM((rows_per_super, d), table_flat.dtype),
            pltpu.SemaphoreType.DMA,
        ],
    )
    def gather_kernel(tbl_hbm, idx_hbm, out_hbm, idx_v, rows_v, sem):
        wid = lax.axis_index("s") * _NC + lax.axis_index("c")
        base = wid * ipw
        pltpu.sync_copy(idx_hbm.at[wid], idx_v)

        @pl.loop(0, supers)
        def _super(sp):
            cps = []
            for j in range(_STREAMS_PER_SUPER):
                g = sp * _STREAMS_PER_SUPER + j
                cps.append(pltpu.async_copy(
                    tbl_hbm.at[idx_v.at[g]],
                    rows_v.at[pl.ds(j * rps, rps)],
                    sem))
            for cp in cps:
                cp.wait()
            pltpu.sync_copy(
                rows_v, out_hbm.at[pl.ds(base + sp * rows_per_super, rows_per_super)])

    return gather_kernel(table_flat, idx)


_VC = 2048
_QR = 512


_CB = 2


_FB = 13


def _transpose_body(tt_ref, out_ref):
    for g in range(tt_ref.shape[0]):
        x = tt_ref[g]
        for c in range(_CB):
            big = jnp.concatenate(
                [x[:, c * _VC + a * _QR: c * _VC + (a + 1) * _QR]
                 for a in range(4)], axis=0)
            out_ref[g, c * _QR:(c + 1) * _QR, :] = jnp.swapaxes(big, 0, 1)


def _tc_transpose(tt):
    f, d, v = tt.shape
    fb = _FB if f % _FB == 0 else 1
    steps = (v + _CB * _VC - 1) // (_CB * _VC)
    chunks = steps * _CB
    return pl.pallas_call(
        _transpose_body,
        grid=(f // fb, steps),
        in_specs=[pl.BlockSpec((fb, d, _CB * _VC), lambda i, j: (i, 0, j))],
        out_specs=pl.BlockSpec((fb, _CB * _QR, 128), lambda i, j: (i, j, 0)),
        out_shape=jax.ShapeDtypeStruct((f, chunks * _QR, 128), jnp.float32),
    )(tt)


def _mlp_body(xi_ref, xe_ref, w1a_ref, w1b_ref, w2_ref, w3_ref,
              b1_ref, b2_ref, b3_ref, o_ref):
    f_cat = xe_ref.shape[0]
    bf = jnp.bfloat16
    z = jnp.dot(xi_ref[...], w1a_ref[...], preferred_element_type=jnp.float32)
    for f in range(f_cat):
        z = z + jnp.dot(xe_ref[f].astype(bf), w1b_ref[f].astype(bf),
                        preferred_element_type=jnp.float32)
    h1 = jnp.maximum(z + b1_ref[...], 0.0)
    h2 = jnp.maximum(
        jnp.dot(h1.astype(bf), w2_ref[...].astype(bf),
                preferred_element_type=jnp.float32) + b2_ref[...], 0.0)
    o = jnp.dot(h2, w3_ref[...], preferred_element_type=jnp.float32) + b3_ref[...]
    o_ref[...] = jax.nn.sigmoid(o)


def _mlp_packed(xi_p, xe_p, w1a_p, w1b_p, w2_p, w3_p, b1_p, b2_p, b3_p,
                tile_r=256):
    r_tot = xi_p.shape[0]
    f_cat = xe_p.shape[0]
    grid = (r_tot // tile_r,)
    full = lambda shape: pl.BlockSpec(shape, lambda i: tuple(0 for _ in shape))
    return pl.pallas_call(
        _mlp_body,
        grid=grid,
        in_specs=[
            pl.BlockSpec((tile_r, xi_p.shape[1]), lambda i: (i, 0)),
            pl.BlockSpec((f_cat, tile_r, 128), lambda i: (0, i, 0)),
            full(w1a_p.shape),
            full(w1b_p.shape),
            full(w2_p.shape),
            full(w3_p.shape),
            full(b1_p.shape),
            full(b2_p.shape),
            full(b3_p.shape),
        ],
        out_specs=pl.BlockSpec((tile_r, 4), lambda i: (i, 0)),
        out_shape=jax.ShapeDtypeStruct((r_tot, 4), jnp.float32),
    )(xi_p, xe_p, w1a_p, w1b_p, w2_p, w3_p, b1_p, b2_p, b3_p)


def kernel(xi, xv, tables, W1, b1, W2, b2, W3, b3):
    b, f_cat = xv.shape
    f, v, d = tables.shape
    f_cont = xi.shape[1]
    tbl_packed = _tc_transpose(jnp.swapaxes(tables, 1, 2))
    rows_pf = tbl_packed.shape[1]
    xvt = xv.astype(jnp.int32).T
    fid = jnp.arange(f, dtype=jnp.int32)[:, None]
    idx = (fid * rows_pf + (xvt >> 11) * _QR + (xvt & (_QR - 1))) * 4 \
        + ((xvt >> 9) & 3)
    ipw = (b * f_cat) // _NW
    idx = idx.reshape(_NW, ipw // _ROWS_PER_STREAM, _ROWS_PER_STREAM)
    xe = _sc_gather(tbl_packed.reshape(f * rows_pf * 4, d), idx)
    xe_p = xe.reshape(f_cat, b // 4, 4 * d)
    xi_p = xi.reshape(b // 4, 4 * f_cont)
    eye4 = jnp.eye(4, dtype=jnp.float32)
    w1a = W1[:f_cont]
    w1b3 = W1[f_cont:].reshape(f_cat, d, -1)
    h1 = W1.shape[1]
    w1a_p = jnp.einsum("xy,cj->xcyj", eye4, w1a).reshape(4 * f_cont, 4 * h1)
    w1b_p = jnp.einsum("xy,fdj->fxdyj", eye4, w1b3).reshape(f_cat, 4 * d, 4 * h1)
    w2_p = jnp.einsum("xy,ij->xiyj", eye4, W2).reshape(4 * h1, 4 * W2.shape[1])
    w3_p = jnp.einsum("xy,j->xjy", eye4, W3[:, 0]).reshape(4 * W3.shape[0], 4)
    b1_p = jnp.tile(b1, 4).reshape(1, -1)
    b2_p = jnp.tile(b2, 4).reshape(1, -1)
    b3_p = b3.reshape(1, 1)
    o_p = _mlp_packed(xi_p, xe_p, w1a_p, w1b_p, w2_p, w3_p, b1_p, b2_p, b3_p)
    return o_p.reshape(b, 1)
